# baseline (speedup 1.0000x reference)
import functools

import jax
import jax.numpy as jnp
from jax import lax
from jax.experimental import pallas as pl
from jax.experimental.pallas import tpu as pltpu
from jax.experimental.pallas import tpu_sc as plsc

C = 19
CP = 20
CROWS = 32
FEAT = 256
N = 131072
NC = 2
NS = 16
NW = NC * NS
CHUNK = 128
LANES = 16
FVREG = FEAT // LANES
ACCW = CP * FEAT
BT = 2048
NB = N // BT

SC_NCHUNK = 28
TOK_PER_W = SC_NCHUNK * CHUNK
SC_TOK = NW * TOK_PER_W
TAIL_B0 = SC_TOK // BT
NTAIL = (N - SC_TOK) // BT
NBT = NB + NTAIL


def _sc_body(f_hbm, p_hbm, osum, ocnt,
             fbuf0, fbuf1, lbuf, acc, cnt, sem0, sem1):
    wid = lax.axis_index("s") * NC + lax.axis_index("c")
    base = wid * TOK_PER_W
    ones = jnp.ones((LANES,), jnp.float32)
    zeros = jnp.zeros((LANES,), jnp.float32)

    def zero_body(r, _):
        o = r * LANES
        acc[pl.ds(o, LANES)] = zeros
        cnt[pl.ds(o, LANES)] = zeros
        return 0

    lax.fori_loop(0, ACCW // LANES, zero_body, 0)

    pltpu.sync_copy(p_hbm.at[pl.ds(base, TOK_PER_W)],
                    lbuf.at[pl.ds(0, TOK_PER_W)])

    def start(ci, buf, sem):
        pltpu.async_copy(f_hbm.at[pl.ds(base + ci * CHUNK, CHUNK)], buf, sem)

    def wait(ci, buf, sem):
        pltpu.make_async_copy(
            f_hbm.at[pl.ds(base + ci * CHUNK, CHUNK)], buf, sem).wait()

    def process(buf, ci):
        @plsc.parallel_loop(0, CHUNK, unroll=4)
        def tok(t):
            lsp = lbuf[pl.ds(ci * CHUNK + t, LANES)]
            off = (lsp * FEAT)[0]
            vals = [buf[t, pl.ds(j * LANES, LANES)] for j in range(FVREG)]
            plsc.addupdate(cnt.at[pl.ds(off, LANES)], ones)
            for j in range(FVREG):
                plsc.addupdate(acc.at[pl.ds(off + j * LANES, LANES)],
                               vals[j])

    start(0, fbuf0, sem0)

    def body2(h, _):
        ci0 = 2 * h
        ci1 = 2 * h + 1
        start(ci1, fbuf1, sem1)
        wait(ci0, fbuf0, sem0)
        process(fbuf0, ci0)

        @pl.when(ci0 + 2 < SC_NCHUNK)
        def _():
            start(ci0 + 2, fbuf0, sem0)

        wait(ci1, fbuf1, sem1)
        process(fbuf1, ci1)
        return 0

    lax.fori_loop(0, SC_NCHUNK // 2, body2, 0)
    pltpu.sync_copy(acc, osum.at[wid])
    pltpu.sync_copy(cnt, ocnt.at[wid])


_sc_partials = functools.partial(
    pl.kernel,
    mesh=plsc.VectorSubcoreMesh(core_axis_name="c", subcore_axis_name="s"),
    out_type=[
        jax.ShapeDtypeStruct((NW, ACCW), jnp.float32),
        jax.ShapeDtypeStruct((NW, ACCW), jnp.float32),
    ],
    scratch_types=[
        pltpu.VMEM((CHUNK, FEAT), jnp.float32),
        pltpu.VMEM((CHUNK, FEAT), jnp.float32),
        pltpu.VMEM((TOK_PER_W + LANES,), jnp.int32),
        pltpu.VMEM((ACCW,), jnp.float32),
        pltpu.VMEM((ACCW,), jnp.float32),
        pltpu.SemaphoreType.DMA,
        pltpu.SemaphoreType.DMA,
    ],
)(_sc_body)


def _centroid_update(cnt19, sums19, cen, amt):
    amount_cxa = jnp.where(cnt19 == 0.0, 1.0, cnt19)
    mean = sums19 / amount_cxa
    denom = cnt19 + amt
    safe = jnp.where(denom == 0.0, 1.0, denom)
    w = jnp.where(cnt19 == 0.0, 0.0, cnt19 / safe)
    return cen * (1.0 - w) + mean * w, amt + cnt19


def _onehot(lbl):
    return (lax.broadcasted_iota(jnp.int32, (CROWS, BT), 0)
            == jnp.broadcast_to(lbl[None, :], (CROWS, BT))).astype(jnp.float32)


def _tc_body(lab1_ref, feat1_ref, lab2_ref, feat2_ref, cen_ref, amt_ref,
             oc_ref, oa_ref, tsum_ref, tcnt_ref, acc1, cnt1, acc2, cnt2):
    i = pl.program_id(0)

    @pl.when(i == 0)
    def _():
        acc1[...] = jnp.zeros((CROWS, FEAT), jnp.float32)
        cnt1[...] = jnp.zeros((CROWS, 128), jnp.float32)
        acc2[...] = jnp.zeros((CROWS, FEAT), jnp.float32)
        cnt2[...] = jnp.zeros((CROWS, 128), jnp.float32)

    @pl.when(i < NB)
    def _():
        oh = _onehot(lab1_ref[0, 0, :])
        acc1[...] += jax.lax.dot_general(
            oh, feat1_ref[...], (((1,), (0,)), ((), ())),
            preferred_element_type=jnp.float32)
        cnt1[...] += jnp.broadcast_to(
            jnp.sum(oh, axis=1, keepdims=True), (CROWS, 128))

    @pl.when(i >= NB)
    def _():
        oh = _onehot(lab2_ref[0, 0, :])
        acc2[...] += jax.lax.dot_general(
            oh, feat2_ref[...], (((1,), (0,)), ((), ())),
            preferred_element_type=jnp.float32)
        cnt2[...] += jnp.broadcast_to(
            jnp.sum(oh, axis=1, keepdims=True), (CROWS, 128))

    @pl.when(i == NB - 1)
    def _():
        oc, oa = _centroid_update(cnt1[0:C, 0:1], acc1[0:C, :],
                                  cen_ref[...], amt_ref[...])
        oc_ref[...] = oc
        oa_ref[...] = oa

    @pl.when(i == NBT - 1)
    def _():
        tsum_ref[...] = acc2[...]
        tcnt_ref[...] = cnt2[...]


def _tc_onehot(lab1, feat1, lab2, feat2, cen, amt):
    return pl.pallas_call(
        _tc_body,
        grid=(NBT,),
        in_specs=[
            pl.BlockSpec((1, 1, BT), lambda i: (jnp.minimum(i, NB - 1), 0, 0)),
            pl.BlockSpec((BT, FEAT), lambda i: (jnp.minimum(i, NB - 1), 0)),
            pl.BlockSpec(
                (1, 1, BT),
                lambda i: (TAIL_B0 + jnp.clip(i - NB, 0, NTAIL - 1), 0, 0)),
            pl.BlockSpec(
                (BT, FEAT),
                lambda i: (TAIL_B0 + jnp.clip(i - NB, 0, NTAIL - 1), 0)),
            pl.BlockSpec((C, FEAT), lambda i: (0, 0)),
            pl.BlockSpec((C, 1), lambda i: (0, 0)),
        ],
        out_specs=[
            pl.BlockSpec((C, FEAT), lambda i: (0, 0)),
            pl.BlockSpec((C, 1), lambda i: (0, 0)),
            pl.BlockSpec((CROWS, FEAT), lambda i: (0, 0)),
            pl.BlockSpec((CROWS, 128), lambda i: (0, 0)),
        ],
        out_shape=[
            jax.ShapeDtypeStruct((C, FEAT), jnp.float32),
            jax.ShapeDtypeStruct((C, 1), jnp.float32),
            jax.ShapeDtypeStruct((CROWS, FEAT), jnp.float32),
            jax.ShapeDtypeStruct((CROWS, 128), jnp.float32),
        ],
        scratch_shapes=[
            pltpu.VMEM((CROWS, FEAT), jnp.float32),
            pltpu.VMEM((CROWS, 128), jnp.float32),
            pltpu.VMEM((CROWS, FEAT), jnp.float32),
            pltpu.VMEM((CROWS, 128), jnp.float32),
        ],
    )(lab1, feat1, lab2, feat2, cen, amt)


def _epilogue(ps, pc, tsum, tcnt, cen, amt, oc_ref, oa_ref):
    sums = ps[0]
    cnts = pc[0]
    for i in range(1, NW):
        sums = sums + ps[i]
        cnts = cnts + pc[i]
    cnt19 = cnts[0:C, 0:1] + tcnt[0:C, 0:1]
    sums19 = sums[0:C, :] + tsum[0:C, :]
    oc, oa = _centroid_update(cnt19, sums19, cen[...], amt[...])
    oc_ref[...] = oc
    oa_ref[...] = oa


def kernel(feat_trs, pred_trs, feat_tgt, pred_tgt,
           Centroid_trs, Amount_trs, Centroid_tgt, Amount_tgt):
    os2, ocn2 = _sc_partials(feat_tgt, pred_tgt)
    a1 = Amount_trs.reshape(C, 1)
    oc1, oa1, tsum, tcnt = _tc_onehot(
        pred_trs.reshape(NB, 1, BT), feat_trs,
        pred_tgt.reshape(NB, 1, BT), feat_tgt,
        Centroid_trs, a1)
    a2 = Amount_tgt.reshape(C, 1)
    oc2, oa2 = pl.pallas_call(
        _epilogue,
        out_shape=[
            jax.ShapeDtypeStruct((C, FEAT), jnp.float32),
            jax.ShapeDtypeStruct((C, 1), jnp.float32),
        ],
    )(os2.reshape(NW, CP, FEAT), ocn2.reshape(NW, CP, FEAT),
      tsum, tcnt, Centroid_tgt, a2)
    return (oc1, oa1.reshape(C), oc2, oa2.reshape(C))


# speedup vs baseline: 1.0014x; 1.0014x over previous
import functools

import jax
import jax.numpy as jnp
from jax import lax
from jax.experimental import pallas as pl
from jax.experimental.pallas import tpu as pltpu
from jax.experimental.pallas import tpu_sc as plsc

C = 19
CP = 20
CROWS = 32
FEAT = 256
N = 131072
NC = 2
NS = 16
NW = NC * NS
CHUNK = 128
LANES = 16
FVREG = FEAT // LANES
ACCW = CP * FEAT
BT = 2048
NB = N // BT

SC_NCHUNK = 28
TOK_PER_W = SC_NCHUNK * CHUNK
SC_TOK = NW * TOK_PER_W
TAIL_B0 = SC_TOK // BT
NTAIL = (N - SC_TOK) // BT
NBT = NB + NTAIL


def _sc_body(f_hbm, p_hbm, osum, ocnt,
             fbuf0, fbuf1, lbuf, acc, cnt, sem0, sem1):
    wid = lax.axis_index("s") * NC + lax.axis_index("c")
    base = wid * TOK_PER_W
    ones = jnp.ones((LANES,), jnp.float32)
    zeros = jnp.zeros((LANES,), jnp.float32)

    def zero_body(r, _):
        o = r * LANES
        acc[pl.ds(o, LANES)] = zeros
        cnt[pl.ds(o, LANES)] = zeros
        return 0

    lax.fori_loop(0, ACCW // LANES, zero_body, 0)

    pltpu.sync_copy(p_hbm.at[pl.ds(base, TOK_PER_W)],
                    lbuf.at[pl.ds(0, TOK_PER_W)])

    def start(ci, buf, sem):
        pltpu.async_copy(f_hbm.at[pl.ds(base + ci * CHUNK, CHUNK)], buf, sem)

    def wait(ci, buf, sem):
        pltpu.make_async_copy(
            f_hbm.at[pl.ds(base + ci * CHUNK, CHUNK)], buf, sem).wait()

    def process(buf, ci):
        def grp(g, _):
            lblv = lbuf[pl.ds(ci * CHUNK + g * LANES, LANES)]
            offv = lblv * FEAT
            tbase = g * LANES

            def load_tok(k):
                return [buf[tbase + k, pl.ds(j * LANES, LANES)]
                        for j in range(FVREG)]

            def store_tok(k, vals):
                off = offv[k]
                plsc.addupdate(cnt.at[pl.ds(off, LANES)], ones)
                for j in range(FVREG):
                    plsc.addupdate(acc.at[pl.ds(off + j * LANES, LANES)],
                                   vals[j])

            vals = load_tok(0)
            for k in range(LANES):
                nxt = load_tok(k + 1) if k + 1 < LANES else None
                store_tok(k, vals)
                vals = nxt
            return 0

        lax.fori_loop(0, CHUNK // LANES, grp, 0)

    start(0, fbuf0, sem0)

    def body2(h, _):
        ci0 = 2 * h
        ci1 = 2 * h + 1
        start(ci1, fbuf1, sem1)
        wait(ci0, fbuf0, sem0)
        process(fbuf0, ci0)

        @pl.when(ci0 + 2 < SC_NCHUNK)
        def _():
            start(ci0 + 2, fbuf0, sem0)

        wait(ci1, fbuf1, sem1)
        process(fbuf1, ci1)
        return 0

    lax.fori_loop(0, SC_NCHUNK // 2, body2, 0)
    pltpu.sync_copy(acc, osum.at[wid])
    pltpu.sync_copy(cnt, ocnt.at[wid])


_sc_partials = functools.partial(
    pl.kernel,
    mesh=plsc.VectorSubcoreMesh(core_axis_name="c", subcore_axis_name="s"),
    out_type=[
        jax.ShapeDtypeStruct((NW, ACCW), jnp.float32),
        jax.ShapeDtypeStruct((NW, ACCW), jnp.float32),
    ],
    scratch_types=[
        pltpu.VMEM((CHUNK, FEAT), jnp.float32),
        pltpu.VMEM((CHUNK, FEAT), jnp.float32),
        pltpu.VMEM((TOK_PER_W + LANES,), jnp.int32),
        pltpu.VMEM((ACCW,), jnp.float32),
        pltpu.VMEM((ACCW,), jnp.float32),
        pltpu.SemaphoreType.DMA,
        pltpu.SemaphoreType.DMA,
    ],
)(_sc_body)


def _centroid_update(cnt19, sums19, cen, amt):
    amount_cxa = jnp.where(cnt19 == 0.0, 1.0, cnt19)
    mean = sums19 / amount_cxa
    denom = cnt19 + amt
    safe = jnp.where(denom == 0.0, 1.0, denom)
    w = jnp.where(cnt19 == 0.0, 0.0, cnt19 / safe)
    return cen * (1.0 - w) + mean * w, amt + cnt19


def _onehot(lbl):
    return (lax.broadcasted_iota(jnp.int32, (CROWS, BT), 0)
            == jnp.broadcast_to(lbl[None, :], (CROWS, BT))).astype(jnp.float32)


def _tc_body(lab1_ref, feat1_ref, lab2_ref, feat2_ref, cen_ref, amt_ref,
             oc_ref, oa_ref, tsum_ref, tcnt_ref, acc1, cnt1, acc2, cnt2):
    i = pl.program_id(0)

    @pl.when(i == 0)
    def _():
        acc1[...] = jnp.zeros((CROWS, FEAT), jnp.float32)
        cnt1[...] = jnp.zeros((CROWS, 128), jnp.float32)
        acc2[...] = jnp.zeros((CROWS, FEAT), jnp.float32)
        cnt2[...] = jnp.zeros((CROWS, 128), jnp.float32)

    @pl.when(i < NB)
    def _():
        oh = _onehot(lab1_ref[0, 0, :])
        acc1[...] += jax.lax.dot_general(
            oh, feat1_ref[...], (((1,), (0,)), ((), ())),
            preferred_element_type=jnp.float32)
        cnt1[...] += jnp.broadcast_to(
            jnp.sum(oh, axis=1, keepdims=True), (CROWS, 128))

    @pl.when(i >= NB)
    def _():
        oh = _onehot(lab2_ref[0, 0, :])
        acc2[...] += jax.lax.dot_general(
            oh, feat2_ref[...], (((1,), (0,)), ((), ())),
            preferred_element_type=jnp.float32)
        cnt2[...] += jnp.broadcast_to(
            jnp.sum(oh, axis=1, keepdims=True), (CROWS, 128))

    @pl.when(i == NB - 1)
    def _():
        oc, oa = _centroid_update(cnt1[0:C, 0:1], acc1[0:C, :],
                                  cen_ref[...], amt_ref[...])
        oc_ref[...] = oc
        oa_ref[...] = oa

    @pl.when(i == NBT - 1)
    def _():
        tsum_ref[...] = acc2[...]
        tcnt_ref[...] = cnt2[...]


def _tc_onehot(lab1, feat1, lab2, feat2, cen, amt):
    return pl.pallas_call(
        _tc_body,
        grid=(NBT,),
        in_specs=[
            pl.BlockSpec((1, 1, BT), lambda i: (jnp.minimum(i, NB - 1), 0, 0)),
            pl.BlockSpec((BT, FEAT), lambda i: (jnp.minimum(i, NB - 1), 0)),
            pl.BlockSpec(
                (1, 1, BT),
                lambda i: (TAIL_B0 + jnp.clip(i - NB, 0, NTAIL - 1), 0, 0)),
            pl.BlockSpec(
                (BT, FEAT),
                lambda i: (TAIL_B0 + jnp.clip(i - NB, 0, NTAIL - 1), 0)),
            pl.BlockSpec((C, FEAT), lambda i: (0, 0)),
            pl.BlockSpec((C, 1), lambda i: (0, 0)),
        ],
        out_specs=[
            pl.BlockSpec((C, FEAT), lambda i: (0, 0)),
            pl.BlockSpec((C, 1), lambda i: (0, 0)),
            pl.BlockSpec((CROWS, FEAT), lambda i: (0, 0)),
            pl.BlockSpec((CROWS, 128), lambda i: (0, 0)),
        ],
        out_shape=[
            jax.ShapeDtypeStruct((C, FEAT), jnp.float32),
            jax.ShapeDtypeStruct((C, 1), jnp.float32),
            jax.ShapeDtypeStruct((CROWS, FEAT), jnp.float32),
            jax.ShapeDtypeStruct((CROWS, 128), jnp.float32),
        ],
        scratch_shapes=[
            pltpu.VMEM((CROWS, FEAT), jnp.float32),
            pltpu.VMEM((CROWS, 128), jnp.float32),
            pltpu.VMEM((CROWS, FEAT), jnp.float32),
            pltpu.VMEM((CROWS, 128), jnp.float32),
        ],
    )(lab1, feat1, lab2, feat2, cen, amt)


def _epilogue(ps, pc, tsum, tcnt, cen, amt, oc_ref, oa_ref):
    sums = ps[0]
    cnts = pc[0]
    for i in range(1, NW):
        sums = sums + ps[i]
        cnts = cnts + pc[i]
    cnt19 = cnts[0:C, 0:1] + tcnt[0:C, 0:1]
    sums19 = sums[0:C, :] + tsum[0:C, :]
    oc, oa = _centroid_update(cnt19, sums19, cen[...], amt[...])
    oc_ref[...] = oc
    oa_ref[...] = oa


def kernel(feat_trs, pred_trs, feat_tgt, pred_tgt,
           Centroid_trs, Amount_trs, Centroid_tgt, Amount_tgt):
    os2, ocn2 = _sc_partials(feat_tgt, pred_tgt)
    a1 = Amount_trs.reshape(C, 1)
    oc1, oa1, tsum, tcnt = _tc_onehot(
        pred_trs.reshape(NB, 1, BT), feat_trs,
        pred_tgt.reshape(NB, 1, BT), feat_tgt,
        Centroid_trs, a1)
    a2 = Amount_tgt.reshape(C, 1)
    oc2, oa2 = pl.pallas_call(
        _epilogue,
        out_shape=[
            jax.ShapeDtypeStruct((C, FEAT), jnp.float32),
            jax.ShapeDtypeStruct((C, 1), jnp.float32),
        ],
    )(os2.reshape(NW, CP, FEAT), ocn2.reshape(NW, CP, FEAT),
      tsum, tcnt, Centroid_tgt, a2)
    return (oc1, oa1.reshape(C), oc2, oa2.reshape(C))


# speedup vs baseline: 1.0329x; 1.0315x over previous
import functools

import jax
import jax.numpy as jnp
from jax import lax
from jax.experimental import pallas as pl
from jax.experimental.pallas import tpu as pltpu
from jax.experimental.pallas import tpu_sc as plsc

C = 19
CP = 20
CROWS = 32
FEAT = 256
N = 131072
NC = 2
NS = 16
NW = NC * NS
CHUNK = 128
LANES = 16
FVREG = FEAT // LANES
ACCW = CP * FEAT
BT = 2048
NB = N // BT

SC_NCHUNK = 30
TOK_PER_W = SC_NCHUNK * CHUNK
SC_TOK = NW * TOK_PER_W
TAIL_B0 = SC_TOK // BT
NTAIL = (N - SC_TOK) // BT
NBT = NB + NTAIL


def _sc_body(f_hbm, p_hbm, osum, ocnt,
             fbuf0, fbuf1, lbuf, acc, cnt, sem0, sem1):
    wid = lax.axis_index("s") * NC + lax.axis_index("c")
    base = wid * TOK_PER_W
    ones = jnp.ones((LANES,), jnp.float32)
    zeros = jnp.zeros((LANES,), jnp.float32)

    def zero_body(r, _):
        o = r * LANES
        acc[pl.ds(o, LANES)] = zeros
        cnt[pl.ds(o, LANES)] = zeros
        return 0

    lax.fori_loop(0, ACCW // LANES, zero_body, 0)

    pltpu.sync_copy(p_hbm.at[pl.ds(base, TOK_PER_W)],
                    lbuf.at[pl.ds(0, TOK_PER_W)])

    def start(ci, buf, sem):
        pltpu.async_copy(f_hbm.at[pl.ds(base + ci * CHUNK, CHUNK)], buf, sem)

    def wait(ci, buf, sem):
        pltpu.make_async_copy(
            f_hbm.at[pl.ds(base + ci * CHUNK, CHUNK)], buf, sem).wait()

    def process(buf, ci):
        def grp(g, _):
            lblv = lbuf[pl.ds(ci * CHUNK + g * LANES, LANES)]
            offv = lblv * FEAT
            tbase = g * LANES

            def load_tok(k):
                return [buf[tbase + k, pl.ds(j * LANES, LANES)]
                        for j in range(FVREG)]

            def store_tok(k, vals):
                off = offv[k]
                plsc.addupdate(cnt.at[pl.ds(off, LANES)], ones)
                for j in range(FVREG):
                    plsc.addupdate(acc.at[pl.ds(off + j * LANES, LANES)],
                                   vals[j])

            vals = load_tok(0)
            for k in range(LANES):
                nxt = load_tok(k + 1) if k + 1 < LANES else None
                store_tok(k, vals)
                vals = nxt
            return 0

        lax.fori_loop(0, CHUNK // LANES, grp, 0)

    start(0, fbuf0, sem0)

    def body2(h, _):
        ci0 = 2 * h
        ci1 = 2 * h + 1
        start(ci1, fbuf1, sem1)
        wait(ci0, fbuf0, sem0)
        process(fbuf0, ci0)

        @pl.when(ci0 + 2 < SC_NCHUNK)
        def _():
            start(ci0 + 2, fbuf0, sem0)

        wait(ci1, fbuf1, sem1)
        process(fbuf1, ci1)
        return 0

    lax.fori_loop(0, SC_NCHUNK // 2, body2, 0)
    pltpu.sync_copy(acc, osum.at[wid])
    pltpu.sync_copy(cnt, ocnt.at[wid])


_sc_partials = functools.partial(
    pl.kernel,
    mesh=plsc.VectorSubcoreMesh(core_axis_name="c", subcore_axis_name="s"),
    out_type=[
        jax.ShapeDtypeStruct((NW, ACCW), jnp.float32),
        jax.ShapeDtypeStruct((NW, ACCW), jnp.float32),
    ],
    scratch_types=[
        pltpu.VMEM((CHUNK, FEAT), jnp.float32),
        pltpu.VMEM((CHUNK, FEAT), jnp.float32),
        pltpu.VMEM((TOK_PER_W + LANES,), jnp.int32),
        pltpu.VMEM((ACCW,), jnp.float32),
        pltpu.VMEM((ACCW,), jnp.float32),
        pltpu.SemaphoreType.DMA,
        pltpu.SemaphoreType.DMA,
    ],
)(_sc_body)


def _centroid_update(cnt19, sums19, cen, amt):
    amount_cxa = jnp.where(cnt19 == 0.0, 1.0, cnt19)
    mean = sums19 / amount_cxa
    denom = cnt19 + amt
    safe = jnp.where(denom == 0.0, 1.0, denom)
    w = jnp.where(cnt19 == 0.0, 0.0, cnt19 / safe)
    return cen * (1.0 - w) + mean * w, amt + cnt19


def _onehot(lbl):
    return (lax.broadcasted_iota(jnp.int32, (CROWS, BT), 0)
            == jnp.broadcast_to(lbl[None, :], (CROWS, BT))).astype(jnp.float32)


def _tc_body(lab1_ref, feat1_ref, lab2_ref, feat2_ref, cen_ref, amt_ref,
             oc_ref, oa_ref, tsum_ref, tcnt_ref, acc1, cnt1, acc2, cnt2):
    i = pl.program_id(0)

    @pl.when(i == 0)
    def _():
        acc1[...] = jnp.zeros((CROWS, FEAT), jnp.float32)
        cnt1[...] = jnp.zeros((CROWS, 128), jnp.float32)
        acc2[...] = jnp.zeros((CROWS, FEAT), jnp.float32)
        cnt2[...] = jnp.zeros((CROWS, 128), jnp.float32)

    @pl.when(i < NB)
    def _():
        oh = _onehot(lab1_ref[0, 0, :])
        acc1[...] += jax.lax.dot_general(
            oh, feat1_ref[...], (((1,), (0,)), ((), ())),
            preferred_element_type=jnp.float32)
        cnt1[...] += jnp.broadcast_to(
            jnp.sum(oh, axis=1, keepdims=True), (CROWS, 128))

    @pl.when(i >= NB)
    def _():
        oh = _onehot(lab2_ref[0, 0, :])
        acc2[...] += jax.lax.dot_general(
            oh, feat2_ref[...], (((1,), (0,)), ((), ())),
            preferred_element_type=jnp.float32)
        cnt2[...] += jnp.broadcast_to(
            jnp.sum(oh, axis=1, keepdims=True), (CROWS, 128))

    @pl.when(i == NB - 1)
    def _():
        oc, oa = _centroid_update(cnt1[0:C, 0:1], acc1[0:C, :],
                                  cen_ref[...], amt_ref[...])
        oc_ref[...] = oc
        oa_ref[...] = oa

    @pl.when(i == NBT - 1)
    def _():
        tsum_ref[...] = acc2[...]
        tcnt_ref[...] = cnt2[...]


def _tc_onehot(lab1, feat1, lab2, feat2, cen, amt):
    return pl.pallas_call(
        _tc_body,
        grid=(NBT,),
        in_specs=[
            pl.BlockSpec((1, 1, BT), lambda i: (jnp.minimum(i, NB - 1), 0, 0)),
            pl.BlockSpec((BT, FEAT), lambda i: (jnp.minimum(i, NB - 1), 0)),
            pl.BlockSpec(
                (1, 1, BT),
                lambda i: (TAIL_B0 + jnp.clip(i - NB, 0, NTAIL - 1), 0, 0)),
            pl.BlockSpec(
                (BT, FEAT),
                lambda i: (TAIL_B0 + jnp.clip(i - NB, 0, NTAIL - 1), 0)),
            pl.BlockSpec((C, FEAT), lambda i: (0, 0)),
            pl.BlockSpec((C, 1), lambda i: (0, 0)),
        ],
        out_specs=[
            pl.BlockSpec((C, FEAT), lambda i: (0, 0)),
            pl.BlockSpec((C, 1), lambda i: (0, 0)),
            pl.BlockSpec((CROWS, FEAT), lambda i: (0, 0)),
            pl.BlockSpec((CROWS, 128), lambda i: (0, 0)),
        ],
        out_shape=[
            jax.ShapeDtypeStruct((C, FEAT), jnp.float32),
            jax.ShapeDtypeStruct((C, 1), jnp.float32),
            jax.ShapeDtypeStruct((CROWS, FEAT), jnp.float32),
            jax.ShapeDtypeStruct((CROWS, 128), jnp.float32),
        ],
        scratch_shapes=[
            pltpu.VMEM((CROWS, FEAT), jnp.float32),
            pltpu.VMEM((CROWS, 128), jnp.float32),
            pltpu.VMEM((CROWS, FEAT), jnp.float32),
            pltpu.VMEM((CROWS, 128), jnp.float32),
        ],
    )(lab1, feat1, lab2, feat2, cen, amt)


def _epilogue(ps, pc, tsum, tcnt, cen, amt, oc_ref, oa_ref):
    sums = ps[0]
    cnts = pc[0]
    for i in range(1, NW):
        sums = sums + ps[i]
        cnts = cnts + pc[i]
    cnt19 = cnts[0:C, 0:1] + tcnt[0:C, 0:1]
    sums19 = sums[0:C, :] + tsum[0:C, :]
    oc, oa = _centroid_update(cnt19, sums19, cen[...], amt[...])
    oc_ref[...] = oc
    oa_ref[...] = oa


def kernel(feat_trs, pred_trs, feat_tgt, pred_tgt,
           Centroid_trs, Amount_trs, Centroid_tgt, Amount_tgt):
    os2, ocn2 = _sc_partials(feat_tgt, pred_tgt)
    a1 = Amount_trs.reshape(C, 1)
    oc1, oa1, tsum, tcnt = _tc_onehot(
        pred_trs.reshape(NB, 1, BT), feat_trs,
        pred_tgt.reshape(NB, 1, BT), feat_tgt,
        Centroid_trs, a1)
    a2 = Amount_tgt.reshape(C, 1)
    oc2, oa2 = pl.pallas_call(
        _epilogue,
        out_shape=[
            jax.ShapeDtypeStruct((C, FEAT), jnp.float32),
            jax.ShapeDtypeStruct((C, 1), jnp.float32),
        ],
    )(os2.reshape(NW, CP, FEAT), ocn2.reshape(NW, CP, FEAT),
      tsum, tcnt, Centroid_tgt, a2)
    return (oc1, oa1.reshape(C), oc2, oa2.reshape(C))
